# Initial kernel scaffold; baseline (speedup 1.0000x reference)
import jax
import jax.numpy as jnp
from jax.experimental import pallas as pl


def kernel(x, edge_index, W1, b1, W2, b2):
    raise NotImplementedError("write your pallas kernel here")


# speedup vs baseline: 3.3579x; 3.3579x over previous
import functools

import jax
import jax.numpy as jnp
from jax import lax
from jax.experimental import pallas as pl
from jax.experimental.pallas import tpu as pltpu
from jax.experimental.pallas import tpu_sc as plsc

N = 10000
E = 320000
D = 128

NC = 2
NS = 16
NW = NC * NS
CH = 128
NCHUNK = 80
EPW = CH * NCHUNK
EP = EPW * NW

ACC_ROWS = 10240
ZROWS = 640
OROWS = 624
OTAIL = N - NS * OROWS

RBLK = 1000


def _spmm_body(sup_hbm, src_hbm, dst_hbm, zeros_hbm, out_hbm,
               src_v, dst_v, buf0, acc, sem0):
    cid = lax.axis_index("c")
    sid = lax.axis_index("s")
    wid = cid * NS + sid

    pltpu.sync_copy(zeros_hbm, acc.at[pl.ds(sid * ZROWS, ZROWS)])
    pltpu.sync_copy(src_hbm.at[wid], src_v)
    pltpu.sync_copy(dst_hbm.at[wid], dst_v)
    plsc.subcore_barrier()

    def step(j, carry):
        pltpu.async_copy(sup_hbm.at[src_v.at[j]], buf0, sem0).wait()
        pltpu.sync_copy(buf0, acc.at[dst_v.at[j]], add=True)
        return carry

    lax.fori_loop(0, NCHUNK, step, 0)

    plsc.subcore_barrier()
    pltpu.sync_copy(acc.at[pl.ds(sid * OROWS, OROWS)],
                    out_hbm.at[pl.ds(cid * N + sid * OROWS, OROWS)])

    @pl.when(sid == NS - 1)
    def _():
        pltpu.sync_copy(acc.at[pl.ds(NS * OROWS, OTAIL)],
                        out_hbm.at[pl.ds(cid * N + NS * OROWS, OTAIL)])


_spmm_sc = pl.kernel(
    _spmm_body,
    out_type=jax.ShapeDtypeStruct((NC * N, D), jnp.float32),
    mesh=plsc.VectorSubcoreMesh(core_axis_name="c", subcore_axis_name="s"),
    scratch_types=[
        pltpu.VMEM((NCHUNK, CH), jnp.int32),
        pltpu.VMEM((NCHUNK, CH), jnp.int32),
        pltpu.VMEM((CH, D), jnp.float32),
        pltpu.VMEM_SHARED((ACC_ROWS, D), jnp.float32),
        pltpu.SemaphoreType.DMA,
    ],
)


def _mm_body(x_ref, w_ref, o_ref):
    o_ref[...] = jnp.dot(x_ref[...], w_ref[...],
                         preferred_element_type=jnp.float32)


def _mid_body(p0_ref, p1_ref, b_ref, w_ref, o_ref):
    h = jnp.maximum(p0_ref[...] + p1_ref[...] + b_ref[...], 0.0)
    o_ref[...] = jnp.dot(h, w_ref[...], preferred_element_type=jnp.float32)


def _fin_body(p0_ref, p1_ref, b_ref, o_ref):
    o_ref[...] = jnp.maximum(p0_ref[...] + p1_ref[...] + b_ref[...], 0.0)


_GRID = N // RBLK
_row_spec = pl.BlockSpec((RBLK, D), lambda i: (i, 0))
_p0_spec = pl.BlockSpec((RBLK, D), lambda i: (i, 0))
_p1_spec = pl.BlockSpec((RBLK, D), lambda i: (i + _GRID, 0))
_b_spec = pl.BlockSpec((1, D), lambda i: (0, 0))
_w_spec = pl.BlockSpec((D, D), lambda i: (0, 0))
_out_f32 = jax.ShapeDtypeStruct((N, D), jnp.float32)

_mm = pl.pallas_call(
    _mm_body, grid=(_GRID,),
    in_specs=[_row_spec, _w_spec], out_specs=_row_spec, out_shape=_out_f32)

_mid = pl.pallas_call(
    _mid_body, grid=(_GRID,),
    in_specs=[_p0_spec, _p1_spec, _b_spec, _w_spec],
    out_specs=_row_spec, out_shape=_out_f32)

_fin = pl.pallas_call(
    _fin_body, grid=(_GRID,),
    in_specs=[_p0_spec, _p1_spec, _b_spec], out_specs=_row_spec,
    out_shape=_out_f32)


def kernel(x, edge_index, W1, b1, W2, b2):
    src = edge_index[0].astype(jnp.int32)
    dst = edge_index[1].astype(jnp.int32)
    pad = EP - E
    src_r = jnp.concatenate(
        [src, jnp.zeros((pad,), jnp.int32)]).reshape(NW, NCHUNK, CH)
    dst_r = jnp.concatenate(
        [dst, jnp.full((pad,), N, jnp.int32)]).reshape(NW, NCHUNK, CH)
    zeros = jnp.zeros((ZROWS, D), jnp.float32)
    b1r = b1.reshape(1, D)
    b2r = b2.reshape(1, D)

    s1 = _mm(x, W1)
    p1 = _spmm_sc(s1, src_r, dst_r, zeros)
    s2 = _mid(p1, p1, b1r, W2)
    p2 = _spmm_sc(s2, src_r, dst_r, zeros)
    return _fin(p2, p2, b2r)
